# baseline (speedup 1.0000x reference)
import functools

import jax
import jax.numpy as jnp
from jax import lax
from jax.experimental import pallas as pl
from jax.experimental.pallas import tpu as pltpu
from jax.experimental.pallas import tpu_sc as plsc

N_NODES = 10000
N_EDGES = 320000
D = 128
D_EDGE = 16

NC = 2
NS = 16
CHUNK = 80
N_CHUNKS = N_EDGES // CHUNK
CH_PER_TILE = N_CHUNKS // NS
HALF = N_NODES // NC
DEG_ROWS = HALF // D + 1
ACC_ROWS = HALF + DEG_ROWS
NL = 16


def _node_matmul(x, W1a, W1b, b1):
    def body(x_ref, wa_ref, wb_ref, b1_ref, a_ref, bb_ref):
        xv = x_ref[...]
        a_ref[...] = jnp.dot(xv, wa_ref[...], preferred_element_type=jnp.float32)
        bb_ref[...] = (jnp.dot(xv, wb_ref[...], preferred_element_type=jnp.float32)
                       + b1_ref[...])
    return pl.pallas_call(
        body,
        out_shape=(jax.ShapeDtypeStruct((N_NODES, D), jnp.float32),
                   jax.ShapeDtypeStruct((N_NODES, D), jnp.float32)),
    )(x, W1a, W1b, b1.reshape(1, D))


_EBLK = 8000


def _edge_matmul(ea, W1c):
    def body(ea_ref, w_ref, c_ref):
        c_ref[...] = jnp.dot(ea_ref[...], w_ref[...],
                             preferred_element_type=jnp.float32)
    return pl.pallas_call(
        body,
        grid=(N_EDGES // _EBLK,),
        in_specs=[pl.BlockSpec((_EBLK, D_EDGE), lambda i: (i, 0)),
                  pl.BlockSpec((D_EDGE, D), lambda i: (0, 0))],
        out_specs=pl.BlockSpec((_EBLK, D), lambda i: (i, 0)),
        out_shape=jax.ShapeDtypeStruct((N_EDGES, D), jnp.float32),
    )(ea, W1c)


_NBLK = 2000


def _final_matmul(S, deg, W2, b2):
    def body(s_ref, d_ref, w_ref, b2_ref, o_ref):
        o_ref[...] = (jnp.dot(s_ref[...], w_ref[...],
                              preferred_element_type=jnp.float32)
                      + d_ref[...] * b2_ref[...])
    return pl.pallas_call(
        body,
        grid=(N_NODES // _NBLK,),
        in_specs=[pl.BlockSpec((_NBLK, D), lambda i: (i, 0)),
                  pl.BlockSpec((_NBLK, 1), lambda i: (i, 0)),
                  pl.BlockSpec((D, D), lambda i: (0, 0)),
                  pl.BlockSpec((1, D), lambda i: (0, 0))],
        out_specs=pl.BlockSpec((_NBLK, D), lambda i: (i, 0)),
        out_shape=jax.ShapeDtypeStruct((N_NODES, D), jnp.float32),
    )(S, deg, W2, b2.reshape(1, D))


def _sc_gather_relu_scatter(A, Bb, C, src, dst):
    mesh = plsc.VectorSubcoreMesh(core_axis_name="c", subcore_axis_name="s")

    @functools.partial(
        pl.kernel,
        mesh=mesh,
        out_type=(jax.ShapeDtypeStruct((N_NODES, D), jnp.float32),
                  jax.ShapeDtypeStruct((NC, DEG_ROWS, D), jnp.float32)),
        scratch_types=(
            [pltpu.VMEM((CHUNK,), jnp.int32)] * 4 +
            [pltpu.VMEM((CHUNK,), jnp.int32)] * 6 +
            [pltpu.VMEM((CHUNK, D), jnp.float32)] * 8 +
            [pltpu.VMEM_SHARED((ACC_ROWS, D), jnp.float32)] +
            [pltpu.SemaphoreType.DMA] * 8
        ),
    )
    def k(a_hbm, b_hbm, c_hbm, src_hbm, dst_hbm, out_hbm, deg_hbm,
          src0, src1, dst0, dst1, sx0, sx1, ib0, ib1, po0, po1,
          a0, a1, b0, b1, c0, c1, m0, m1, s_sh,
          gse0, gse1, gc0, gc1, gab0, gab1, sca0, sca1):
        cid = lax.axis_index("c")
        sid = lax.axis_index("s")
        sets = ((src0, dst0, sx0, ib0, po0, a0, b0, c0, m0, gse0, gc0, gab0, sca0),
                (src1, dst1, sx1, ib1, po1, a1, b1, c1, m1, gse1, gc1, gab1, sca1))

        zero16 = jnp.zeros((NL,), jnp.float32)
        lane = lax.iota(jnp.int32, NL)
        lanes = [lane + NL * j for j in range(D // NL)]

        def zrow_body(e, carry):
            for j in range(D // NL):
                m0[e, pl.ds(j * NL, NL)] = zero16
            return carry
        lax.fori_loop(0, CHUNK, zrow_body, 0)
        for off, rows in ((0, 80), (80, 80), (160, 80), (240, 75)):
            pltpu.sync_copy(m0.at[pl.ds(0, rows)],
                            s_sh.at[pl.ds(sid * 315 + off, rows)])

        plsc.subcore_barrier()

        lo = cid * HALF
        lo2 = (1 - cid) * HALF

        def chunk_base(ch):
            return (sid + ch * NS) * CHUNK

        def fetch_stage1(ch, bufs):
            src_v, dst_v, c_v = bufs[0], bufs[1], bufs[7]
            gse, gc = bufs[9], bufs[10]
            base = chunk_base(ch)
            pltpu.async_copy(src_hbm.at[pl.ds(base, CHUNK)], src_v, gse)
            pltpu.async_copy(dst_hbm.at[pl.ds(base, CHUNK)], dst_v, gse)
            pltpu.async_copy(c_hbm.at[pl.ds(base, CHUNK)], c_v, gc)

        def wait_stage1(ch, bufs):
            src_v, dst_v, gse = bufs[0], bufs[1], bufs[9]
            base = chunk_base(ch)
            pltpu.make_async_copy(
                src_hbm.at[pl.ds(base, CHUNK)], src_v, gse).wait()
            pltpu.make_async_copy(
                dst_hbm.at[pl.ds(base, CHUNK)], dst_v, gse).wait()

        def fetch_stage2(bufs):
            src_v, dst_v = bufs[0], bufs[1]
            a_v, b_v, gab = bufs[5], bufs[6], bufs[11]
            pltpu.async_copy(a_hbm.at[src_v], a_v, gab)
            pltpu.async_copy(b_hbm.at[dst_v], b_v, gab)

        fetch_stage1(0, sets[0])
        wait_stage1(0, sets[0])
        fetch_stage2(sets[0])

        def loop_body(kk, carry):
            for b in range(2):
                ch = 2 * kk + b
                (src_v, dst_v, sidx_v, inb_v, pos_v,
                 a_v, b_v, c_v, m_v, gse, gc, gab, sca) = sets[b]
                nxt = sets[1 - b]

                @pl.when(ch < CH_PER_TILE - 1)
                def _():
                    fetch_stage1(ch + 1, nxt)

                base = chunk_base(ch)
                pltpu.make_async_copy(a_hbm.at[src_v], a_v, gab).wait()
                pltpu.make_async_copy(b_hbm.at[dst_v], b_v, gab).wait()
                pltpu.make_async_copy(
                    c_hbm.at[pl.ds(base, CHUNK)], c_v, gc).wait()

                @pl.when(kk > 0)
                def _():
                    pltpu.make_async_copy(m_v, s_sh.at[sidx_v], sca).wait()

                for g in range(CHUNK // NL):
                    d16 = dst_v[pl.ds(g * NL, NL)]
                    r16 = d16 - lo
                    in_half = (r16 >= 0) & (r16 < HALF)
                    r2 = d16 - lo2
                    sidx_v[pl.ds(g * NL, NL)] = jnp.where(
                        in_half, r16, HALF + lax.shift_right_logical(r2, 7))
                    inb_v[pl.ds(g * NL, NL)] = jnp.where(in_half, 1, 0)
                    pos_v[pl.ds(g * NL, NL)] = lax.bitwise_and(r2, 127)

                def group_body(g, c2):
                    f16 = inb_v[pl.ds(g * NL, NL)]
                    p16 = pos_v[pl.ds(g * NL, NL)]
                    for l in range(NL):
                        e = g * NL + l
                        flag = f16[l] > 0
                        pos = p16[l]
                        for j in range(D // NL):
                            sl = pl.ds(j * NL, NL)
                            msg = jnp.maximum(
                                a_v[e, sl] + b_v[e, sl] + c_v[e, sl], 0.0)
                            oh = jnp.where(lanes[j] == pos, 1.0, 0.0)
                            m_v[e, sl] = jnp.where(flag, msg, oh)
                    return c2
                lax.fori_loop(0, CHUNK // NL, group_body, 0)

                pltpu.async_copy(m_v, s_sh.at[sidx_v], sca, add=True)

                @pl.when(ch < CH_PER_TILE - 1)
                def _():
                    wait_stage1(ch + 1, nxt)
                    fetch_stage2(nxt)
            return carry
        lax.fori_loop(0, CH_PER_TILE // 2, loop_body, 0)

        for b in range(2):
            m_v, sidx_v, sca = sets[b][8], sets[b][2], sets[b][12]
            pltpu.make_async_copy(m_v, s_sh.at[sidx_v], sca).wait()

        plsc.subcore_barrier()

        @pl.when(sid == 0)
        def _():
            pltpu.sync_copy(s_sh.at[pl.ds(0, HALF)],
                            out_hbm.at[pl.ds(cid * HALF, HALF)])
            pltpu.sync_copy(s_sh.at[pl.ds(HALF, DEG_ROWS)], deg_hbm.at[cid])

    return k(A, Bb, C, src, dst)


def kernel(x, edge_index, edge_attr, W1, b1, W2, b2):
    src = edge_index[0].astype(jnp.int32)
    dst = edge_index[1].astype(jnp.int32)
    W1a = W1[:D]
    W1b = W1[D:2 * D]
    W1c = W1[2 * D:]

    A, Bb = _node_matmul(x, W1a, W1b, b1)
    C = _edge_matmul(edge_attr, W1c)
    S, degr = _sc_gather_relu_scatter(A, Bb, C, src, dst)

    deg = jnp.concatenate([degr[1].reshape(-1)[:HALF],
                           degr[0].reshape(-1)[:HALF]])
    return _final_matmul(S, deg.reshape(N_NODES, 1), W2, b2)


# speedup vs baseline: 1.4641x; 1.4641x over previous
import functools

import jax
import jax.numpy as jnp
from jax import lax
from jax.experimental import pallas as pl
from jax.experimental.pallas import tpu as pltpu
from jax.experimental.pallas import tpu_sc as plsc

N_NODES = 10000
N_EDGES = 320000
D = 128
D_EDGE = 16

NC = 2
NS = 16
CHUNK = 80
N_CHUNKS = N_EDGES // CHUNK
CH_PER_TILE = N_CHUNKS // NS
HALF = N_NODES // NC
DEG_ROWS = HALF // D + 1
ACC_ROWS = HALF + DEG_ROWS
NL = 16


def _node_matmul(x, W1a, W1b, b1):
    def body(x_ref, wa_ref, wb_ref, b1_ref, a_ref, bb_ref):
        xv = x_ref[...]
        a_ref[...] = jnp.dot(xv, wa_ref[...], preferred_element_type=jnp.float32)
        bb_ref[...] = (jnp.dot(xv, wb_ref[...], preferred_element_type=jnp.float32)
                       + b1_ref[...])
    return pl.pallas_call(
        body,
        out_shape=(jax.ShapeDtypeStruct((N_NODES, D), jnp.float32),
                   jax.ShapeDtypeStruct((N_NODES, D), jnp.float32)),
    )(x, W1a, W1b, b1.reshape(1, D))


_EBLK = 8000


def _edge_matmul(ea, W1c):
    def body(ea_ref, w_ref, c_ref):
        c_ref[...] = jnp.dot(ea_ref[...], w_ref[...],
                             preferred_element_type=jnp.float32)
    return pl.pallas_call(
        body,
        grid=(N_EDGES // _EBLK,),
        in_specs=[pl.BlockSpec((_EBLK, D_EDGE), lambda i: (i, 0)),
                  pl.BlockSpec((D_EDGE, D), lambda i: (0, 0))],
        out_specs=pl.BlockSpec((_EBLK, D), lambda i: (i, 0)),
        out_shape=jax.ShapeDtypeStruct((N_EDGES, D), jnp.float32),
    )(ea, W1c)


_NBLK = 2000


def _final_matmul(S, deg, W2, b2):
    def body(s_ref, d_ref, w_ref, b2_ref, o_ref):
        o_ref[...] = (jnp.dot(s_ref[...], w_ref[...],
                              preferred_element_type=jnp.float32)
                      + d_ref[...] * b2_ref[...])
    return pl.pallas_call(
        body,
        grid=(N_NODES // _NBLK,),
        in_specs=[pl.BlockSpec((_NBLK, D), lambda i: (i, 0)),
                  pl.BlockSpec((_NBLK, 1), lambda i: (i, 0)),
                  pl.BlockSpec((D, D), lambda i: (0, 0)),
                  pl.BlockSpec((1, D), lambda i: (0, 0))],
        out_specs=pl.BlockSpec((_NBLK, D), lambda i: (i, 0)),
        out_shape=jax.ShapeDtypeStruct((N_NODES, D), jnp.float32),
    )(S, deg, W2, b2.reshape(1, D))


def _sc_gather_relu_scatter(A, Bb, C, src, dst):
    mesh = plsc.VectorSubcoreMesh(core_axis_name="c", subcore_axis_name="s")

    @functools.partial(
        pl.kernel,
        mesh=mesh,
        out_type=(jax.ShapeDtypeStruct((N_NODES, D), jnp.float32),
                  jax.ShapeDtypeStruct((NC, DEG_ROWS, D), jnp.float32)),
        scratch_types=(
            [pltpu.VMEM((CHUNK,), jnp.int32)] * 8 +
            [pltpu.VMEM((CHUNK,), jnp.int32)] * 8 +
            [pltpu.VMEM((CHUNK, D), jnp.float32)] * 8 +
            [pltpu.VMEM_SHARED((ACC_ROWS, D), jnp.float32)] +
            [pltpu.SemaphoreType.DMA] * 4 +
            [pltpu.SemaphoreType.DMA] * 6
        ),
    )
    def k(a_hbm, b_hbm, c_hbm, src_hbm, dst_hbm, out_hbm, deg_hbm,
          src0, src1, src2, src3, dst0, dst1, dst2, dst3,
          sx0, sx1, ib0, ib1, po0, po1, dd0, dd1,
          a0, a1, b0, b1, c0, c1, m0, m1, s_sh,
          gse0, gse1, gse2, gse3, gc0, gc1, gab0, gab1, sca0, sca1):
        cid = lax.axis_index("c")
        sid = lax.axis_index("s")
        idxs = ((src0, dst0, gse0), (src1, dst1, gse1),
                (src2, dst2, gse2), (src3, dst3, gse3))
        rows = ((sx0, ib0, po0, dd0, a0, b0, c0, m0, gc0, gab0, sca0),
                (sx1, ib1, po1, dd1, a1, b1, c1, m1, gc1, gab1, sca1))

        zero16 = jnp.zeros((NL,), jnp.float32)
        lane = lax.iota(jnp.int32, NL)
        lanes = [lane + NL * j for j in range(D // NL)]

        def zrow_body(e, carry):
            for j in range(D // NL):
                m0[e, pl.ds(j * NL, NL)] = zero16
            return carry
        lax.fori_loop(0, CHUNK, zrow_body, 0)
        for off, nrows in ((0, 80), (80, 80), (160, 80), (240, 75)):
            pltpu.sync_copy(m0.at[pl.ds(0, nrows)],
                            s_sh.at[pl.ds(sid * 315 + off, nrows)])

        plsc.subcore_barrier()

        lo = cid * HALF
        lo2 = (1 - cid) * HALF

        def chunk_base(ch):
            return (sid + ch * NS) * CHUNK

        def issue_idx(ch, q):
            src_v, dst_v, gse = idxs[q]
            base = chunk_base(ch)
            pltpu.async_copy(src_hbm.at[pl.ds(base, CHUNK)], src_v, gse)
            pltpu.async_copy(dst_hbm.at[pl.ds(base, CHUNK)], dst_v, gse)

        def wait_idx(ch, q):
            src_v, dst_v, gse = idxs[q]
            base = chunk_base(ch)
            pltpu.make_async_copy(
                src_hbm.at[pl.ds(base, CHUNK)], src_v, gse).wait()
            pltpu.make_async_copy(
                dst_hbm.at[pl.ds(base, CHUNK)], dst_v, gse).wait()

        def issue_rows(ch, q, rb):
            src_v, dst_v, _ = idxs[q]
            a_v, b_v, c_v = rb[4], rb[5], rb[6]
            gc, gab = rb[8], rb[9]
            pltpu.async_copy(a_hbm.at[src_v], a_v, gab)
            pltpu.async_copy(b_hbm.at[dst_v], b_v, gab)
            pltpu.async_copy(
                c_hbm.at[pl.ds(chunk_base(ch), CHUNK)], c_v, gc)

        issue_idx(0, 0)
        issue_idx(1, 1)
        wait_idx(0, 0)
        issue_rows(0, 0, rows[0])

        def loop_body(kk, carry):
            for b in range(2):
                ch = 2 * kk + b
                (sidx_v, inb_v, pos_v, ddst_v,
                 a_v, b_v, c_v, m_v, gc, gab, sca) = rows[b]

                for par in range(2):
                    qcur = b + 2 * par

                    @pl.when(lax.rem(kk, 2) == par)
                    def _(qcur=qcur, ch=ch):
                        @pl.when(ch + 1 < CH_PER_TILE)
                        def _():
                            qn = (qcur + 1) % 4
                            wait_idx(ch + 1, qn)
                            issue_rows(ch + 1, qn, rows[1 - b])

                        @pl.when(ch + 2 < CH_PER_TILE)
                        def _():
                            issue_idx(ch + 2, (qcur + 2) % 4)

                        src_v, dst_v, _ = idxs[qcur]
                        pltpu.make_async_copy(
                            a_hbm.at[src_v], a_v, gab).wait()
                        pltpu.make_async_copy(
                            b_hbm.at[dst_v], b_v, gab).wait()
                        pltpu.make_async_copy(
                            c_hbm.at[pl.ds(chunk_base(ch), CHUNK)],
                            c_v, gc).wait()

                        for g in range(CHUNK // NL):
                            ddst_v[pl.ds(g * NL, NL)] = \
                                dst_v[pl.ds(g * NL, NL)]

                @pl.when(kk > 0)
                def _():
                    pltpu.make_async_copy(m_v, s_sh.at[sidx_v], sca).wait()

                for g in range(CHUNK // NL):
                    d16 = ddst_v[pl.ds(g * NL, NL)]
                    r16 = d16 - lo
                    in_half = (r16 >= 0) & (r16 < HALF)
                    r2 = d16 - lo2
                    sidx_v[pl.ds(g * NL, NL)] = jnp.where(
                        in_half, r16,
                        HALF + lax.shift_right_logical(r2, 7))
                    inb_v[pl.ds(g * NL, NL)] = jnp.where(in_half, 1, 0)
                    pos_v[pl.ds(g * NL, NL)] = lax.bitwise_and(r2, 127)

                def group_body(g, c2):
                    f16 = inb_v[pl.ds(g * NL, NL)]
                    p16 = pos_v[pl.ds(g * NL, NL)]
                    for l in range(NL):
                        e = g * NL + l
                        flag = f16[l] > 0
                        pos = p16[l]
                        for j in range(D // NL):
                            sl = pl.ds(j * NL, NL)
                            msg = jnp.maximum(
                                a_v[e, sl] + b_v[e, sl] + c_v[e, sl], 0.0)
                            oh = jnp.where(lanes[j] == pos, 1.0, 0.0)
                            m_v[e, sl] = jnp.where(flag, msg, oh)
                    return c2
                lax.fori_loop(0, CHUNK // NL, group_body, 0)

                pltpu.async_copy(m_v, s_sh.at[sidx_v], sca, add=True)
            return carry
        lax.fori_loop(0, CH_PER_TILE // 2, loop_body, 0)

        for b in range(2):
            sidx_v, m_v, sca = rows[b][0], rows[b][7], rows[b][10]
            pltpu.make_async_copy(m_v, s_sh.at[sidx_v], sca).wait()

        plsc.subcore_barrier()

        @pl.when(sid == 0)
        def _():
            pltpu.sync_copy(s_sh.at[pl.ds(0, HALF)],
                            out_hbm.at[pl.ds(cid * HALF, HALF)])
            pltpu.sync_copy(s_sh.at[pl.ds(HALF, DEG_ROWS)], deg_hbm.at[cid])

    return k(A, Bb, C, src, dst)


def kernel(x, edge_index, edge_attr, W1, b1, W2, b2):
    src = edge_index[0].astype(jnp.int32)
    dst = edge_index[1].astype(jnp.int32)
    W1a = W1[:D]
    W1b = W1[D:2 * D]
    W1c = W1[2 * D:]

    A, Bb = _node_matmul(x, W1a, W1b, b1)
    C = _edge_matmul(edge_attr, W1c)
    S, degr = _sc_gather_relu_scatter(A, Bb, C, src, dst)

    deg = jnp.concatenate([degr[1].reshape(-1)[:HALF],
                           degr[0].reshape(-1)[:HALF]])
    return _final_matmul(S, deg.reshape(N_NODES, 1), W2, b2)


# speedup vs baseline: 1.4713x; 1.0049x over previous
import functools

import jax
import jax.numpy as jnp
from jax import lax
from jax.experimental import pallas as pl
from jax.experimental.pallas import tpu as pltpu
from jax.experimental.pallas import tpu_sc as plsc

N_NODES = 10000
N_EDGES = 320000
D = 128
D_EDGE = 16

NC = 2
NS = 16
CHUNK = 80
N_CHUNKS = N_EDGES // CHUNK
CH_PER_TILE = N_CHUNKS // NS
HALF = N_NODES // NC
DEG_ROWS = HALF // D + 1
ACC_ROWS = HALF + DEG_ROWS
NL = 16


def _node_matmul(x, W1a, W1b, b1):
    def body(x_ref, wa_ref, wb_ref, b1_ref, a_ref, bb_ref):
        xv = x_ref[...]
        a_ref[...] = jnp.dot(xv, wa_ref[...], preferred_element_type=jnp.float32)
        bb_ref[...] = (jnp.dot(xv, wb_ref[...], preferred_element_type=jnp.float32)
                       + b1_ref[...])
    return pl.pallas_call(
        body,
        out_shape=(jax.ShapeDtypeStruct((N_NODES, D), jnp.float32),
                   jax.ShapeDtypeStruct((N_NODES, D), jnp.float32)),
    )(x, W1a, W1b, b1.reshape(1, D))


_EBLK = 8000


def _edge_matmul(ea, W1c):
    def body(ea_ref, w_ref, c_ref):
        c_ref[...] = jnp.dot(ea_ref[...], w_ref[...],
                             preferred_element_type=jnp.float32)
    return pl.pallas_call(
        body,
        grid=(N_EDGES // _EBLK,),
        in_specs=[pl.BlockSpec((_EBLK, D_EDGE), lambda i: (i, 0)),
                  pl.BlockSpec((D_EDGE, D), lambda i: (0, 0))],
        out_specs=pl.BlockSpec((_EBLK, D), lambda i: (i, 0)),
        out_shape=jax.ShapeDtypeStruct((N_EDGES, D), jnp.float32),
    )(ea, W1c)


_NBLK = 2000


def _final_matmul(S, deg, W2, b2):
    def body(s_ref, d_ref, w_ref, b2_ref, o_ref):
        o_ref[...] = (jnp.dot(s_ref[...], w_ref[...],
                              preferred_element_type=jnp.float32)
                      + d_ref[...] * b2_ref[...])
    return pl.pallas_call(
        body,
        grid=(N_NODES // _NBLK,),
        in_specs=[pl.BlockSpec((_NBLK, D), lambda i: (i, 0)),
                  pl.BlockSpec((_NBLK, 1), lambda i: (i, 0)),
                  pl.BlockSpec((D, D), lambda i: (0, 0)),
                  pl.BlockSpec((1, D), lambda i: (0, 0))],
        out_specs=pl.BlockSpec((_NBLK, D), lambda i: (i, 0)),
        out_shape=jax.ShapeDtypeStruct((N_NODES, D), jnp.float32),
    )(S, deg, W2, b2.reshape(1, D))


def _sc_gather_relu_scatter(A, Bb, C, src, dst):
    mesh = plsc.VectorSubcoreMesh(core_axis_name="c", subcore_axis_name="s")

    @functools.partial(
        pl.kernel,
        mesh=mesh,
        out_type=(jax.ShapeDtypeStruct((N_NODES, D), jnp.float32),
                  jax.ShapeDtypeStruct((NC, DEG_ROWS, D), jnp.float32)),
        scratch_types=(
            [pltpu.VMEM((CHUNK,), jnp.int32)] * 8 +
            [pltpu.VMEM((CHUNK,), jnp.int32)] * 8 +
            [pltpu.VMEM((CHUNK, D), jnp.float32)] * 8 +
            [pltpu.VMEM_SHARED((ACC_ROWS, D), jnp.float32)] +
            [pltpu.SemaphoreType.DMA] * 4 +
            [pltpu.SemaphoreType.DMA] * 6
        ),
    )
    def k(a_hbm, b_hbm, c_hbm, src_hbm, dst_hbm, out_hbm, deg_hbm,
          src0, src1, src2, src3, dst0, dst1, dst2, dst3,
          sx0, sx1, ib0, ib1, po0, po1, dd0, dd1,
          a0, a1, b0, b1, c0, c1, m0, m1, s_sh,
          gse0, gse1, gse2, gse3, gc0, gc1, gab0, gab1, sca0, sca1):
        cid = lax.axis_index("c")
        sid = lax.axis_index("s")
        idxs = ((src0, dst0, gse0), (src1, dst1, gse1),
                (src2, dst2, gse2), (src3, dst3, gse3))
        rows = ((sx0, ib0, po0, dd0, a0, b0, c0, m0, gc0, gab0, sca0),
                (sx1, ib1, po1, dd1, a1, b1, c1, m1, gc1, gab1, sca1))

        zero16 = jnp.zeros((NL,), jnp.float32)
        lane = lax.iota(jnp.int32, NL)
        lanes = [lane + NL * j for j in range(D // NL)]

        def zrow_body(e, carry):
            for j in range(D // NL):
                m0[e, pl.ds(j * NL, NL)] = zero16
            return carry
        lax.fori_loop(0, CHUNK, zrow_body, 0)
        for off, nrows in ((0, 80), (80, 80), (160, 80), (240, 75)):
            pltpu.sync_copy(m0.at[pl.ds(0, nrows)],
                            s_sh.at[pl.ds(sid * 315 + off, nrows)])

        plsc.subcore_barrier()

        lo = cid * HALF
        lo2 = (1 - cid) * HALF

        def chunk_base(ch):
            return (sid + ch * NS) * CHUNK

        def issue_idx(ch, q):
            src_v, dst_v, gse = idxs[q]
            base = chunk_base(ch)
            pltpu.async_copy(src_hbm.at[pl.ds(base, CHUNK)], src_v, gse)
            pltpu.async_copy(dst_hbm.at[pl.ds(base, CHUNK)], dst_v, gse)

        def wait_idx(ch, q):
            src_v, dst_v, gse = idxs[q]
            base = chunk_base(ch)
            pltpu.make_async_copy(
                src_hbm.at[pl.ds(base, CHUNK)], src_v, gse).wait()
            pltpu.make_async_copy(
                dst_hbm.at[pl.ds(base, CHUNK)], dst_v, gse).wait()

        def issue_rows(ch, q, rb):
            src_v, dst_v, _ = idxs[q]
            a_v, b_v, c_v = rb[4], rb[5], rb[6]
            gc, gab = rb[8], rb[9]
            pltpu.async_copy(a_hbm.at[src_v], a_v, gab)
            pltpu.async_copy(b_hbm.at[dst_v], b_v, gab)
            pltpu.async_copy(
                c_hbm.at[pl.ds(chunk_base(ch), CHUNK)], c_v, gc)

        issue_idx(0, 0)
        issue_idx(1, 1)
        wait_idx(0, 0)
        issue_rows(0, 0, rows[0])

        def loop_body(kk, carry):
            for b in range(2):
                ch = 2 * kk + b
                (sidx_v, inb_v, pos_v, ddst_v,
                 a_v, b_v, c_v, m_v, gc, gab, sca) = rows[b]

                for par in range(2):
                    qcur = b + 2 * par

                    @pl.when(lax.rem(kk, 2) == par)
                    def _(qcur=qcur, ch=ch):
                        @pl.when(ch + 1 < CH_PER_TILE)
                        def _():
                            qn = (qcur + 1) % 4
                            wait_idx(ch + 1, qn)
                            issue_rows(ch + 1, qn, rows[1 - b])

                        @pl.when(ch + 2 < CH_PER_TILE)
                        def _():
                            issue_idx(ch + 2, (qcur + 2) % 4)

                        src_v, dst_v, _ = idxs[qcur]
                        pltpu.make_async_copy(
                            a_hbm.at[src_v], a_v, gab).wait()
                        pltpu.make_async_copy(
                            b_hbm.at[dst_v], b_v, gab).wait()
                        pltpu.make_async_copy(
                            c_hbm.at[pl.ds(chunk_base(ch), CHUNK)],
                            c_v, gc).wait()

                        for g in range(CHUNK // NL):
                            ddst_v[pl.ds(g * NL, NL)] = \
                                dst_v[pl.ds(g * NL, NL)]

                @pl.when(kk > 0)
                def _():
                    pltpu.make_async_copy(m_v, s_sh.at[sidx_v], sca).wait()

                for g in range(CHUNK // NL):
                    d16 = ddst_v[pl.ds(g * NL, NL)]
                    r16 = d16 - lo
                    in_half = (r16 >= 0) & (r16 < HALF)
                    r2 = d16 - lo2
                    sidx_v[pl.ds(g * NL, NL)] = jnp.where(
                        in_half, r16,
                        HALF + lax.shift_right_logical(r2, 7))
                    inb_v[pl.ds(g * NL, NL)] = jnp.where(in_half, 1, 0)
                    pos_v[pl.ds(g * NL, NL)] = lax.bitwise_and(r2, 127)

                def group_body(g, c2):
                    f16 = inb_v[pl.ds(g * NL, NL)].astype(jnp.float32)
                    p16 = pos_v[pl.ds(g * NL, NL)]
                    for l in range(NL):
                        e = g * NL + l
                        flagf = f16[l]
                        pos = p16[l]
                        for j in range(D // NL):
                            sl = pl.ds(j * NL, NL)
                            m_v[e, sl] = jnp.maximum(
                                a_v[e, sl] + b_v[e, sl] + c_v[e, sl],
                                0.0) * flagf
                        poff = lax.bitwise_and(pos, 112)
                        osl = pl.ds(poff, NL)
                        ohv = jnp.where(lane == lax.bitwise_and(pos, 15),
                                        1.0 - flagf, 0.0)
                        m_v[e, osl] = m_v[e, osl] + ohv
                    return c2
                lax.fori_loop(0, CHUNK // NL, group_body, 0)

                pltpu.async_copy(m_v, s_sh.at[sidx_v], sca, add=True)
            return carry
        lax.fori_loop(0, CH_PER_TILE // 2, loop_body, 0)

        for b in range(2):
            sidx_v, m_v, sca = rows[b][0], rows[b][7], rows[b][10]
            pltpu.make_async_copy(m_v, s_sh.at[sidx_v], sca).wait()

        plsc.subcore_barrier()

        @pl.when(sid == 0)
        def _():
            pltpu.sync_copy(s_sh.at[pl.ds(0, HALF)],
                            out_hbm.at[pl.ds(cid * HALF, HALF)])
            pltpu.sync_copy(s_sh.at[pl.ds(HALF, DEG_ROWS)], deg_hbm.at[cid])

    return k(A, Bb, C, src, dst)


def kernel(x, edge_index, edge_attr, W1, b1, W2, b2):
    src = edge_index[0].astype(jnp.int32)
    dst = edge_index[1].astype(jnp.int32)
    W1a = W1[:D]
    W1b = W1[D:2 * D]
    W1c = W1[2 * D:]

    A, Bb = _node_matmul(x, W1a, W1b, b1)
    C = _edge_matmul(edge_attr, W1c)
    S, degr = _sc_gather_relu_scatter(A, Bb, C, src, dst)

    deg = jnp.concatenate([degr[1].reshape(-1)[:HALF],
                           degr[0].reshape(-1)[:HALF]])
    return _final_matmul(S, deg.reshape(N_NODES, 1), W2, b2)


# speedup vs baseline: 1.4722x; 1.0006x over previous
import functools

import jax
import jax.numpy as jnp
from jax import lax
from jax.experimental import pallas as pl
from jax.experimental.pallas import tpu as pltpu
from jax.experimental.pallas import tpu_sc as plsc

N_NODES = 10000
N_EDGES = 320000
D = 128
D_EDGE = 16

NC = 2
NS = 16
CHUNK = 80
N_CHUNKS = N_EDGES // CHUNK
CH_PER_TILE = N_CHUNKS // NS
HALF = N_NODES // NC
DEG_ROWS = HALF // D + 1
ACC_ROWS = HALF + DEG_ROWS
NL = 16


def _node_matmul(x, W1a, W1b, b1):
    def body(x_ref, wa_ref, wb_ref, b1_ref, a_ref, bb_ref):
        xv = x_ref[...]
        a_ref[...] = jnp.dot(xv, wa_ref[...], preferred_element_type=jnp.float32)
        bb_ref[...] = (jnp.dot(xv, wb_ref[...], preferred_element_type=jnp.float32)
                       + b1_ref[...])
    return pl.pallas_call(
        body,
        out_shape=(jax.ShapeDtypeStruct((N_NODES, D), jnp.float32),
                   jax.ShapeDtypeStruct((N_NODES, D), jnp.float32)),
    )(x, W1a, W1b, b1.reshape(1, D))


_EBLK = 16000


def _edge_matmul(ea, W1c):
    def body(ea_ref, w_ref, c_ref):
        c_ref[...] = jnp.dot(ea_ref[...], w_ref[...],
                             preferred_element_type=jnp.float32)
    return pl.pallas_call(
        body,
        grid=(N_EDGES // _EBLK,),
        in_specs=[pl.BlockSpec((_EBLK, D_EDGE), lambda i: (i, 0)),
                  pl.BlockSpec((D_EDGE, D), lambda i: (0, 0))],
        out_specs=pl.BlockSpec((_EBLK, D), lambda i: (i, 0)),
        out_shape=jax.ShapeDtypeStruct((N_EDGES, D), jnp.float32),
    )(ea, W1c)


_NBLK = 2000


def _final_matmul(S, deg, W2, b2):
    def body(s_ref, d_ref, w_ref, b2_ref, o_ref):
        o_ref[...] = (jnp.dot(s_ref[...], w_ref[...],
                              preferred_element_type=jnp.float32)
                      + d_ref[...] * b2_ref[...])
    return pl.pallas_call(
        body,
        grid=(N_NODES // _NBLK,),
        in_specs=[pl.BlockSpec((_NBLK, D), lambda i: (i, 0)),
                  pl.BlockSpec((_NBLK, 1), lambda i: (i, 0)),
                  pl.BlockSpec((D, D), lambda i: (0, 0)),
                  pl.BlockSpec((1, D), lambda i: (0, 0))],
        out_specs=pl.BlockSpec((_NBLK, D), lambda i: (i, 0)),
        out_shape=jax.ShapeDtypeStruct((N_NODES, D), jnp.float32),
    )(S, deg, W2, b2.reshape(1, D))


def _sc_gather_relu_scatter(A, Bb, C, src, dst):
    mesh = plsc.VectorSubcoreMesh(core_axis_name="c", subcore_axis_name="s")

    @functools.partial(
        pl.kernel,
        mesh=mesh,
        out_type=(jax.ShapeDtypeStruct((N_NODES, D), jnp.float32),
                  jax.ShapeDtypeStruct((NC, DEG_ROWS, D), jnp.float32)),
        scratch_types=(
            [pltpu.VMEM((CHUNK,), jnp.int32)] * 8 +
            [pltpu.VMEM((CHUNK,), jnp.int32)] * 8 +
            [pltpu.VMEM((CHUNK, D), jnp.float32)] * 8 +
            [pltpu.VMEM_SHARED((ACC_ROWS, D), jnp.float32)] +
            [pltpu.SemaphoreType.DMA] * 4 +
            [pltpu.SemaphoreType.DMA] * 6
        ),
    )
    def k(a_hbm, b_hbm, c_hbm, src_hbm, dst_hbm, out_hbm, deg_hbm,
          src0, src1, src2, src3, dst0, dst1, dst2, dst3,
          sx0, sx1, ib0, ib1, po0, po1, dd0, dd1,
          a0, a1, b0, b1, c0, c1, m0, m1, s_sh,
          gse0, gse1, gse2, gse3, gc0, gc1, gab0, gab1, sca0, sca1):
        cid = lax.axis_index("c")
        sid = lax.axis_index("s")
        idxs = ((src0, dst0, gse0), (src1, dst1, gse1),
                (src2, dst2, gse2), (src3, dst3, gse3))
        rows = ((sx0, ib0, po0, dd0, a0, b0, c0, m0, gc0, gab0, sca0),
                (sx1, ib1, po1, dd1, a1, b1, c1, m1, gc1, gab1, sca1))

        zero16 = jnp.zeros((NL,), jnp.float32)
        lane = lax.iota(jnp.int32, NL)
        lanes = [lane + NL * j for j in range(D // NL)]

        def zrow_body(e, carry):
            for j in range(D // NL):
                m0[e, pl.ds(j * NL, NL)] = zero16
            return carry
        lax.fori_loop(0, CHUNK, zrow_body, 0)
        for off, nrows in ((0, 80), (80, 80), (160, 80), (240, 75)):
            pltpu.sync_copy(m0.at[pl.ds(0, nrows)],
                            s_sh.at[pl.ds(sid * 315 + off, nrows)])

        plsc.subcore_barrier()

        lo = cid * HALF
        lo2 = (1 - cid) * HALF

        def chunk_base(ch):
            return (sid + ch * NS) * CHUNK

        def issue_idx(ch, q):
            src_v, dst_v, gse = idxs[q]
            base = chunk_base(ch)
            pltpu.async_copy(src_hbm.at[pl.ds(base, CHUNK)], src_v, gse)
            pltpu.async_copy(dst_hbm.at[pl.ds(base, CHUNK)], dst_v, gse)

        def wait_idx(ch, q):
            src_v, dst_v, gse = idxs[q]
            base = chunk_base(ch)
            pltpu.make_async_copy(
                src_hbm.at[pl.ds(base, CHUNK)], src_v, gse).wait()
            pltpu.make_async_copy(
                dst_hbm.at[pl.ds(base, CHUNK)], dst_v, gse).wait()

        def issue_rows(ch, q, rb):
            src_v, dst_v, _ = idxs[q]
            a_v, b_v, c_v = rb[4], rb[5], rb[6]
            gc, gab = rb[8], rb[9]
            pltpu.async_copy(a_hbm.at[src_v], a_v, gab)
            pltpu.async_copy(b_hbm.at[dst_v], b_v, gab)
            pltpu.async_copy(
                c_hbm.at[pl.ds(chunk_base(ch), CHUNK)], c_v, gc)

        issue_idx(0, 0)
        issue_idx(1, 1)
        wait_idx(0, 0)
        issue_rows(0, 0, rows[0])

        def loop_body(kk, carry):
            for b in range(2):
                ch = 2 * kk + b
                (sidx_v, inb_v, pos_v, ddst_v,
                 a_v, b_v, c_v, m_v, gc, gab, sca) = rows[b]

                for par in range(2):
                    qcur = b + 2 * par

                    @pl.when(lax.rem(kk, 2) == par)
                    def _(qcur=qcur, ch=ch):
                        @pl.when(ch + 1 < CH_PER_TILE)
                        def _():
                            qn = (qcur + 1) % 4
                            wait_idx(ch + 1, qn)
                            issue_rows(ch + 1, qn, rows[1 - b])

                        @pl.when(ch + 2 < CH_PER_TILE)
                        def _():
                            issue_idx(ch + 2, (qcur + 2) % 4)

                        src_v, dst_v, _ = idxs[qcur]
                        pltpu.make_async_copy(
                            a_hbm.at[src_v], a_v, gab).wait()
                        pltpu.make_async_copy(
                            b_hbm.at[dst_v], b_v, gab).wait()
                        pltpu.make_async_copy(
                            c_hbm.at[pl.ds(chunk_base(ch), CHUNK)],
                            c_v, gc).wait()

                        for g in range(CHUNK // NL):
                            ddst_v[pl.ds(g * NL, NL)] = \
                                dst_v[pl.ds(g * NL, NL)]

                @pl.when(kk > 0)
                def _():
                    pltpu.make_async_copy(m_v, s_sh.at[sidx_v], sca).wait()

                for g in range(CHUNK // NL):
                    d16 = ddst_v[pl.ds(g * NL, NL)]
                    r16 = d16 - lo
                    in_half = (r16 >= 0) & (r16 < HALF)
                    r2 = d16 - lo2
                    sidx_v[pl.ds(g * NL, NL)] = jnp.where(
                        in_half, r16,
                        HALF + lax.shift_right_logical(r2, 7))
                    inb_v[pl.ds(g * NL, NL)] = jnp.where(in_half, 1, 0)
                    pos_v[pl.ds(g * NL, NL)] = lax.bitwise_and(r2, 127)

                def group_body(g, c2):
                    f16 = inb_v[pl.ds(g * NL, NL)].astype(jnp.float32)
                    p16 = pos_v[pl.ds(g * NL, NL)]
                    for l in range(NL):
                        e = g * NL + l
                        flagf = f16[l]
                        pos = p16[l]
                        for j in range(D // NL):
                            sl = pl.ds(j * NL, NL)
                            m_v[e, sl] = jnp.maximum(
                                a_v[e, sl] + b_v[e, sl] + c_v[e, sl],
                                0.0) * flagf
                        poff = lax.bitwise_and(pos, 112)
                        osl = pl.ds(poff, NL)
                        ohv = jnp.where(lane == lax.bitwise_and(pos, 15),
                                        1.0 - flagf, 0.0)
                        m_v[e, osl] = m_v[e, osl] + ohv
                    return c2
                lax.fori_loop(0, CHUNK // NL, group_body, 0)

                pltpu.async_copy(m_v, s_sh.at[sidx_v], sca, add=True)
            return carry
        lax.fori_loop(0, CH_PER_TILE // 2, loop_body, 0)

        for b in range(2):
            sidx_v, m_v, sca = rows[b][0], rows[b][7], rows[b][10]
            pltpu.make_async_copy(m_v, s_sh.at[sidx_v], sca).wait()

        plsc.subcore_barrier()

        @pl.when(sid == 0)
        def _():
            pltpu.sync_copy(s_sh.at[pl.ds(0, HALF)],
                            out_hbm.at[pl.ds(cid * HALF, HALF)])
            pltpu.sync_copy(s_sh.at[pl.ds(HALF, DEG_ROWS)], deg_hbm.at[cid])

    return k(A, Bb, C, src, dst)


def kernel(x, edge_index, edge_attr, W1, b1, W2, b2):
    src = edge_index[0].astype(jnp.int32)
    dst = edge_index[1].astype(jnp.int32)
    W1a = W1[:D]
    W1b = W1[D:2 * D]
    W1c = W1[2 * D:]

    A, Bb = _node_matmul(x, W1a, W1b, b1)
    C = _edge_matmul(edge_attr, W1c)
    S, degr = _sc_gather_relu_scatter(A, Bb, C, src, dst)

    deg = jnp.concatenate([degr[1].reshape(-1)[:HALF],
                           degr[0].reshape(-1)[:HALF]])
    return _final_matmul(S, deg.reshape(N_NODES, 1), W2, b2)


# speedup vs baseline: 1.4788x; 1.0045x over previous
import functools

import jax
import jax.numpy as jnp
from jax import lax
from jax.experimental import pallas as pl
from jax.experimental.pallas import tpu as pltpu
from jax.experimental.pallas import tpu_sc as plsc

N_NODES = 10000
N_EDGES = 320000
D = 128
D_EDGE = 16

NC = 2
NS = 16
CHUNK = 80
N_CHUNKS = N_EDGES // CHUNK
CH_PER_TILE = N_CHUNKS // NS
HALF = N_NODES // NC
DEG_ROWS = HALF // D + 1
ACC_ROWS = HALF + DEG_ROWS
NL = 16


_EBLK = 12800
_XBLK = N_NODES // (N_EDGES // _EBLK)


def _pre_matmuls(x, ea, W1a, W1b, W1c, b1):
    def body(ea_ref, x_ref, wc_ref, wa_ref, wb_ref, b1_ref,
             c_ref, a_ref, bb_ref):
        c_ref[...] = jnp.dot(ea_ref[...], wc_ref[...],
                             preferred_element_type=jnp.float32)
        xv = x_ref[...]
        a_ref[...] = jnp.dot(xv, wa_ref[...], preferred_element_type=jnp.float32)
        bb_ref[...] = (jnp.dot(xv, wb_ref[...], preferred_element_type=jnp.float32)
                       + b1_ref[...])
    return pl.pallas_call(
        body,
        grid=(N_EDGES // _EBLK,),
        in_specs=[pl.BlockSpec((_EBLK, D_EDGE), lambda i: (i, 0)),
                  pl.BlockSpec((_XBLK, D), lambda i: (i, 0)),
                  pl.BlockSpec((D_EDGE, D), lambda i: (0, 0)),
                  pl.BlockSpec((D, D), lambda i: (0, 0)),
                  pl.BlockSpec((D, D), lambda i: (0, 0)),
                  pl.BlockSpec((1, D), lambda i: (0, 0))],
        out_specs=(pl.BlockSpec((_EBLK, D), lambda i: (i, 0)),
                   pl.BlockSpec((_XBLK, D), lambda i: (i, 0)),
                   pl.BlockSpec((_XBLK, D), lambda i: (i, 0))),
        out_shape=(jax.ShapeDtypeStruct((N_EDGES, D), jnp.float32),
                   jax.ShapeDtypeStruct((N_NODES, D), jnp.float32),
                   jax.ShapeDtypeStruct((N_NODES, D), jnp.float32)),
    )(ea, x, W1c, W1a, W1b, b1.reshape(1, D))


_NBLK = 2000


def _final_matmul(S, deg, W2, b2):
    def body(s_ref, d_ref, w_ref, b2_ref, o_ref):
        o_ref[...] = (jnp.dot(s_ref[...], w_ref[...],
                              preferred_element_type=jnp.float32)
                      + d_ref[...] * b2_ref[...])
    return pl.pallas_call(
        body,
        grid=(N_NODES // _NBLK,),
        in_specs=[pl.BlockSpec((_NBLK, D), lambda i: (i, 0)),
                  pl.BlockSpec((_NBLK, 1), lambda i: (i, 0)),
                  pl.BlockSpec((D, D), lambda i: (0, 0)),
                  pl.BlockSpec((1, D), lambda i: (0, 0))],
        out_specs=pl.BlockSpec((_NBLK, D), lambda i: (i, 0)),
        out_shape=jax.ShapeDtypeStruct((N_NODES, D), jnp.float32),
    )(S, deg, W2, b2.reshape(1, D))


def _sc_gather_relu_scatter(A, Bb, C, src, dst):
    mesh = plsc.VectorSubcoreMesh(core_axis_name="c", subcore_axis_name="s")

    @functools.partial(
        pl.kernel,
        mesh=mesh,
        out_type=(jax.ShapeDtypeStruct((N_NODES, D), jnp.float32),
                  jax.ShapeDtypeStruct((NC, DEG_ROWS, D), jnp.float32)),
        scratch_types=(
            [pltpu.VMEM((CHUNK,), jnp.int32)] * 8 +
            [pltpu.VMEM((CHUNK,), jnp.int32)] * 8 +
            [pltpu.VMEM((CHUNK, D), jnp.float32)] * 8 +
            [pltpu.VMEM_SHARED((ACC_ROWS, D), jnp.float32)] +
            [pltpu.SemaphoreType.DMA] * 4 +
            [pltpu.SemaphoreType.DMA] * 6
        ),
    )
    def k(a_hbm, b_hbm, c_hbm, src_hbm, dst_hbm, out_hbm, deg_hbm,
          src0, src1, src2, src3, dst0, dst1, dst2, dst3,
          sx0, sx1, ib0, ib1, po0, po1, dd0, dd1,
          a0, a1, b0, b1, c0, c1, m0, m1, s_sh,
          gse0, gse1, gse2, gse3, gc0, gc1, gab0, gab1, sca0, sca1):
        cid = lax.axis_index("c")
        sid = lax.axis_index("s")
        idxs = ((src0, dst0, gse0), (src1, dst1, gse1),
                (src2, dst2, gse2), (src3, dst3, gse3))
        rows = ((sx0, ib0, po0, dd0, a0, b0, c0, m0, gc0, gab0, sca0),
                (sx1, ib1, po1, dd1, a1, b1, c1, m1, gc1, gab1, sca1))

        zero16 = jnp.zeros((NL,), jnp.float32)
        lane = lax.iota(jnp.int32, NL)
        lanes = [lane + NL * j for j in range(D // NL)]

        def zrow_body(e, carry):
            for j in range(D // NL):
                m0[e, pl.ds(j * NL, NL)] = zero16
            return carry
        lax.fori_loop(0, CHUNK, zrow_body, 0)
        for off, nrows in ((0, 80), (80, 80), (160, 80), (240, 75)):
            pltpu.sync_copy(m0.at[pl.ds(0, nrows)],
                            s_sh.at[pl.ds(sid * 315 + off, nrows)])

        plsc.subcore_barrier()

        lo = cid * HALF
        lo2 = (1 - cid) * HALF

        def chunk_base(ch):
            return (sid + ch * NS) * CHUNK

        def issue_idx(ch, q):
            src_v, dst_v, gse = idxs[q]
            base = chunk_base(ch)
            pltpu.async_copy(src_hbm.at[pl.ds(base, CHUNK)], src_v, gse)
            pltpu.async_copy(dst_hbm.at[pl.ds(base, CHUNK)], dst_v, gse)

        def wait_idx(ch, q):
            src_v, dst_v, gse = idxs[q]
            base = chunk_base(ch)
            pltpu.make_async_copy(
                src_hbm.at[pl.ds(base, CHUNK)], src_v, gse).wait()
            pltpu.make_async_copy(
                dst_hbm.at[pl.ds(base, CHUNK)], dst_v, gse).wait()

        def issue_rows(ch, q, rb):
            src_v, dst_v, _ = idxs[q]
            a_v, b_v, c_v = rb[4], rb[5], rb[6]
            gc, gab = rb[8], rb[9]
            pltpu.async_copy(a_hbm.at[src_v], a_v, gab)
            pltpu.async_copy(b_hbm.at[dst_v], b_v, gab)
            pltpu.async_copy(
                c_hbm.at[pl.ds(chunk_base(ch), CHUNK)], c_v, gc)

        issue_idx(0, 0)
        issue_idx(1, 1)
        wait_idx(0, 0)
        issue_rows(0, 0, rows[0])

        def loop_body(kk, carry):
            for b in range(2):
                ch = 2 * kk + b
                (sidx_v, inb_v, pos_v, ddst_v,
                 a_v, b_v, c_v, m_v, gc, gab, sca) = rows[b]

                for par in range(2):
                    qcur = b + 2 * par

                    @pl.when(lax.rem(kk, 2) == par)
                    def _(qcur=qcur, ch=ch):
                        @pl.when(ch + 1 < CH_PER_TILE)
                        def _():
                            qn = (qcur + 1) % 4
                            wait_idx(ch + 1, qn)
                            issue_rows(ch + 1, qn, rows[1 - b])

                        @pl.when(ch + 2 < CH_PER_TILE)
                        def _():
                            issue_idx(ch + 2, (qcur + 2) % 4)

                        src_v, dst_v, _ = idxs[qcur]
                        pltpu.make_async_copy(
                            a_hbm.at[src_v], a_v, gab).wait()
                        pltpu.make_async_copy(
                            b_hbm.at[dst_v], b_v, gab).wait()
                        pltpu.make_async_copy(
                            c_hbm.at[pl.ds(chunk_base(ch), CHUNK)],
                            c_v, gc).wait()

                        for g in range(CHUNK // NL):
                            ddst_v[pl.ds(g * NL, NL)] = \
                                dst_v[pl.ds(g * NL, NL)]

                @pl.when(kk > 0)
                def _():
                    pltpu.make_async_copy(m_v, s_sh.at[sidx_v], sca).wait()

                for g in range(CHUNK // NL):
                    d16 = ddst_v[pl.ds(g * NL, NL)]
                    r16 = d16 - lo
                    in_half = (r16 >= 0) & (r16 < HALF)
                    r2 = d16 - lo2
                    sidx_v[pl.ds(g * NL, NL)] = jnp.where(
                        in_half, r16,
                        HALF + lax.shift_right_logical(r2, 7))
                    inb_v[pl.ds(g * NL, NL)] = jnp.where(in_half, 1, 0)
                    pos_v[pl.ds(g * NL, NL)] = lax.bitwise_and(r2, 127)

                def group_body(g, c2):
                    f16 = inb_v[pl.ds(g * NL, NL)].astype(jnp.float32)
                    p16 = pos_v[pl.ds(g * NL, NL)]
                    for l in range(NL):
                        e = g * NL + l
                        flagf = f16[l]
                        pos = p16[l]
                        for j in range(D // NL):
                            sl = pl.ds(j * NL, NL)
                            m_v[e, sl] = jnp.maximum(
                                a_v[e, sl] + b_v[e, sl] + c_v[e, sl],
                                0.0) * flagf
                        poff = lax.bitwise_and(pos, 112)
                        osl = pl.ds(poff, NL)
                        ohv = jnp.where(lane == lax.bitwise_and(pos, 15),
                                        1.0 - flagf, 0.0)
                        m_v[e, osl] = m_v[e, osl] + ohv
                    return c2
                lax.fori_loop(0, CHUNK // NL, group_body, 0)

                pltpu.async_copy(m_v, s_sh.at[sidx_v], sca, add=True)
            return carry
        lax.fori_loop(0, CH_PER_TILE // 2, loop_body, 0)

        for b in range(2):
            sidx_v, m_v, sca = rows[b][0], rows[b][7], rows[b][10]
            pltpu.make_async_copy(m_v, s_sh.at[sidx_v], sca).wait()

        plsc.subcore_barrier()

        @pl.when(sid == 0)
        def _():
            pltpu.sync_copy(s_sh.at[pl.ds(0, HALF)],
                            out_hbm.at[pl.ds(cid * HALF, HALF)])
            pltpu.sync_copy(s_sh.at[pl.ds(HALF, DEG_ROWS)], deg_hbm.at[cid])

    return k(A, Bb, C, src, dst)


def kernel(x, edge_index, edge_attr, W1, b1, W2, b2):
    src = edge_index[0].astype(jnp.int32)
    dst = edge_index[1].astype(jnp.int32)
    W1a = W1[:D]
    W1b = W1[D:2 * D]
    W1c = W1[2 * D:]

    C, A, Bb = _pre_matmuls(x, edge_attr, W1a, W1b, W1c, b1)
    S, degr = _sc_gather_relu_scatter(A, Bb, C, src, dst)

    deg = jnp.concatenate([degr[1].reshape(-1)[:HALF],
                           degr[0].reshape(-1)[:HALF]])
    return _final_matmul(S, deg.reshape(N_NODES, 1), W2, b2)
